# baseline (speedup 1.0000x reference)
import functools

import jax
import jax.numpy as jnp
from jax import lax
from jax.experimental import pallas as pl
from jax.experimental.pallas import tpu as pltpu
from jax.experimental.pallas import tpu_sc as plsc

N = 8388608
VOCAB = 100000
NC = 2
NS = 16
L = 16
NW = NC * NS
VP = 102400
S = VP // NW
SLABS = S // L
SP = 3264
VOUT = NW * SP
NWCH = N // NW
CH = 4096
NCHUNK = NWCH // CH

_mesh = plsc.VectorSubcoreMesh(core_axis_name="c", subcore_axis_name="s")
_params = pltpu.CompilerParams(needs_layout_passes=False)


def _i32(v):
    return jnp.int32(v)


def _fori(n, body, init):
    return lax.fori_loop(jnp.int32(0), jnp.int32(n) if isinstance(n, int) else n,
                         body, init)


def _wid():
    return lax.axis_index("c") * _i32(NS) + lax.axis_index("s")


_GDN = lax.GatherDimensionNumbers(
    offset_dims=(), collapsed_slice_dims=(0,), start_index_map=(0,))


def _dyngather(vec, idx):
    return lax.gather(vec, idx[:, None], _GDN, (1,),
                      mode=lax.GatherScatterMode.PROMISE_IN_BOUNDS)


def _splat(vec, lane):
    return _dyngather(vec, jnp.full((L,), lane, jnp.int32))


@functools.partial(
    pl.kernel,
    out_type=jax.ShapeDtypeStruct((NW, VP), jnp.int32),
    mesh=_mesh,
    compiler_params=_params,
    scratch_types=[
        pltpu.VMEM((VP,), jnp.int32),
        pltpu.VMEM((2, CH), jnp.int32),
        pltpu.SemaphoreType.DMA,
        pltpu.SemaphoreType.DMA,
    ],
)
def _hist_kernel(x_hbm, part_hbm, hist_v, xbuf, sem0, sem1):
    w = _wid()
    sems = (sem0, sem1)
    zero = jnp.zeros((L,), jnp.int32)

    def zbody(i, _):
        hist_v[pl.ds(i * _i32(L), L)] = zero
        return _

    _fori(VP // L, zbody, 0)

    base = w * _i32(NWCH)

    base = w * _i32(NWCH)

    def xcopy(c, b):
        return pltpu.make_async_copy(
            x_hbm.at[pl.ds(base + c * _i32(CH), CH)], xbuf.at[_i32(b)], sems[b])

    xcopy(_i32(0), 0).start()

    def outer(g, _):
        for b in range(2):
            c = g * _i32(2) + _i32(b)
            nxt = c + _i32(1)

            @pl.when(nxt < _i32(NCHUNK))
            def _start():
                xcopy(nxt, 1 - b).start()

            xcopy(c, b).wait()

            def slab(k, _2):
                k8 = k * _i32(8 * L)
                vs = [xbuf[_i32(b), pl.ds(k8 + _i32(u * L), L)] for u in range(8)]
                cms = [plsc.scan_count(v) for v in vs]
                for v, (cnt, last) in zip(vs, cms):
                    plsc.addupdate_scatter(hist_v, [v], cnt, mask=last)
                return _2

            _fori(CH // L // 8, slab, 0)
        return _

    _fori(NCHUNK // 2, outer, 0)
    pltpu.sync_copy(hist_v, part_hbm.at[w])


@functools.partial(
    pl.kernel,
    out_type=(
        jax.ShapeDtypeStruct((VP,), jnp.int32),
        jax.ShapeDtypeStruct((NW, L), jnp.int32),
    ),
    mesh=_mesh,
    compiler_params=_params,
    scratch_types=[
        pltpu.VMEM((NW, S), jnp.int32),
        pltpu.VMEM((S,), jnp.int32),
        pltpu.VMEM((L,), jnp.int32),
        pltpu.SemaphoreType.DMA,
    ],
)
def _merge_kernel(part_hbm, hist_hbm, tot_hbm, rows_v, acc_v, stage_v, sem):
    w = _wid()
    descs = []
    for j in range(NW):
        d = pltpu.make_async_copy(
            part_hbm.at[_i32(j), pl.ds(w * _i32(S), S)], rows_v.at[_i32(j)], sem)
        d.start()
        descs.append(d)
    for d in descs:
        d.wait()

    def slab(k, occ_tot):
        a = rows_v[_i32(0), pl.ds(k * _i32(L), L)]
        for j in range(1, NW):
            a = a + rows_v[_i32(j), pl.ds(k * _i32(L), L)]
        acc_v[pl.ds(k * _i32(L), L)] = a
        return occ_tot + (a > 0).astype(jnp.int32)

    occ_tot = _fori(SLABS, slab, jnp.zeros((L,), jnp.int32))
    total = jnp.sum(occ_tot, dtype=jnp.int32)
    stage_v[...] = jnp.zeros((L,), jnp.int32) + total
    pltpu.sync_copy(acc_v, hist_hbm.at[pl.ds(w * _i32(S), S)])
    pltpu.sync_copy(stage_v, tot_hbm.at[w])


@functools.partial(
    pl.kernel,
    out_type=(
        jax.ShapeDtypeStruct((VP,), jnp.int32),
        jax.ShapeDtypeStruct((NW, S), jnp.int32),
        jax.ShapeDtypeStruct((NW, S), jnp.int32),
    ),
    mesh=_mesh,
    compiler_params=_params,
    scratch_types=[
        pltpu.VMEM((NW, L), jnp.int32),
        pltpu.VMEM((S,), jnp.int32),
        pltpu.VMEM((S,), jnp.int32),
        pltpu.VMEM((S + L,), jnp.int32),
        pltpu.VMEM((S + L,), jnp.int32),
    ],
)
def _rank_kernel(hist_hbm, tot_hbm, rank_hbm, pv_hbm, pc_hbm,
                 tot_v, hs_v, rk_v, cv_v, cc_v):
    w = _wid()
    pltpu.sync_copy(tot_hbm, tot_v)
    pltpu.sync_copy(hist_hbm.at[pl.ds(w * _i32(S), S)], hs_v)

    zvec = jnp.zeros((L,), jnp.int32)
    off = zvec
    for j in range(NW):
        row = tot_v[_i32(j)]
        off = off + jnp.where(jnp.full((L,), j, jnp.int32) < w, row, zvec)

    iota = lax.iota(jnp.int32, L)
    vbase = w * _i32(S) + iota

    def slab(k, carry):
        h = hs_v[pl.ds(k * _i32(L), L)]
        occ = h > 0
        occ_i = occ.astype(jnp.int32)
        lidx = plsc.cumsum(occ_i)
        rank = off + (carry - _i32(1)) + lidx
        rk_v[pl.ds(k * _i32(L), L)] = rank
        vglob = vbase + k * _i32(L)
        plsc.store_compressed(cv_v.at[pl.ds(carry, L)], vglob, mask=occ)
        plsc.store_compressed(cc_v.at[pl.ds(carry, L)], h, mask=occ)
        return carry + jnp.sum(occ_i, dtype=jnp.int32)

    _fori(SLABS, slab, _i32(0))

    pltpu.sync_copy(rk_v, rank_hbm.at[pl.ds(w * _i32(S), S)])
    pltpu.sync_copy(cv_v.at[pl.ds(_i32(0), S)], pv_hbm.at[w])
    pltpu.sync_copy(cc_v.at[pl.ds(_i32(0), S)], pc_hbm.at[w])


@functools.partial(
    pl.kernel,
    out_type=(
        jax.ShapeDtypeStruct((VOUT,), jnp.int32),
        jax.ShapeDtypeStruct((VOUT,), jnp.int32),
    ),
    mesh=_mesh,
    compiler_params=_params,
    scratch_types=[
        pltpu.VMEM((NW, L), jnp.int32),
        pltpu.VMEM((NW * S,), jnp.int32),
        pltpu.VMEM((SP,), jnp.int32),
        pltpu.VMEM((SP,), jnp.int32),
        pltpu.SemaphoreType.DMA,
    ],
)
def _final_kernel(tot_hbm, pv_hbm, pc_hbm, vals_hbm, cnts_hbm,
                  tot_v, buf_v, src_v, out_v, sem):
    w = _wid()
    pltpu.sync_copy(tot_hbm, tot_v)

    iota = lax.iota(jnp.int32, L)
    zvec = jnp.zeros((L,), jnp.int32)
    off_lo = zvec
    off_hi = zvec
    for j in range(NW):
        row = tot_v[_i32(j)]
        if j < 15:
            off_lo = off_lo + jnp.where(iota > j, row, zvec)
        if j < 16:
            off_hi = off_hi + row
        else:
            off_hi = off_hi + jnp.where(iota > j - 16, row, zvec)
    t31 = tot_v[_i32(NW - 1)]
    nuq = _splat(off_hi, 15) + t31

    b0 = w * _i32(SP)

    bnd = [_splat(off_lo, k) for k in range(1, 16)] + \
          [_splat(off_hi, k) for k in range(16)] + [nuq]

    def mapslab(k, _):
        r = b0 + k * _i32(L) + iota
        p = zvec
        for bk in bnd:
            p = p + (bk <= r).astype(jnp.int32)
        pcl = jnp.minimum(p, _i32(NW - 1))
        ia = jnp.minimum(pcl, _i32(15))
        ib = jnp.clip(pcl - _i32(16), _i32(0), _i32(15))
        offp = jnp.where(pcl < _i32(16),
                         _dyngather(off_lo, ia),
                         _dyngather(off_hi, ib))
        src = pcl * _i32(S) + (r - offp)
        src = jnp.clip(src, _i32(0), _i32(NW * S - 1))
        src_v[pl.ds(k * _i32(L), L)] = src
        return _

    _fori(SP // L, mapslab, 0)

    for src_hbm, dst_hbm in ((pv_hbm, vals_hbm), (pc_hbm, cnts_hbm)):
        descs = []
        for i in range(NW):
            j = (w + _i32(i)) % _i32(NW)
            d = pltpu.make_async_copy(
                src_hbm.at[j], buf_v.at[pl.ds(j * _i32(S), S)], sem)
            d.start()
            descs.append(d)
        for d in descs:
            d.wait()

        def gslab(k, _):
            r = b0 + k * _i32(L) + iota
            idx = src_v[pl.ds(k * _i32(L), L)]
            g = plsc.load_gather(buf_v, [idx])
            out_v[pl.ds(k * _i32(L), L)] = jnp.where(r < nuq, g, zvec)
            return _

        _fori(SP // L, gslab, 0)
        pltpu.sync_copy(out_v, dst_hbm.at[pl.ds(b0, SP)])


@functools.partial(
    pl.kernel,
    out_type=jax.ShapeDtypeStruct((N,), jnp.int32),
    mesh=_mesh,
    compiler_params=_params,
    scratch_types=[
        pltpu.VMEM((VP,), jnp.int32),
        pltpu.VMEM((2, CH), jnp.int32),
        pltpu.VMEM((2, CH), jnp.int32),
        pltpu.SemaphoreType.DMA,
        pltpu.SemaphoreType.DMA,
        pltpu.SemaphoreType.DMA,
        pltpu.SemaphoreType.DMA,
        pltpu.SemaphoreType.DMA,
    ],
)
def _inverse_kernel(x_hbm, rank_hbm, inv_hbm, rank_v, xbuf, ybuf,
                    xs0, xs1, ys0, ys1, rsem):
    w = _wid()
    rdescs = []
    for i in range(NW):
        j = (w + _i32(i)) % _i32(NW)
        d = pltpu.make_async_copy(
            rank_hbm.at[pl.ds(j * _i32(S), S)], rank_v.at[pl.ds(j * _i32(S), S)],
            rsem)
        d.start()
        rdescs.append(d)

    base = w * _i32(NWCH)
    xsems = (xs0, xs1)
    ysems = (ys0, ys1)

    def xcopy(c, b):
        return pltpu.make_async_copy(
            x_hbm.at[pl.ds(base + c * _i32(CH), CH)], xbuf.at[_i32(b)], xsems[b])

    def ycopy(c, b):
        return pltpu.make_async_copy(
            ybuf.at[_i32(b)], inv_hbm.at[pl.ds(base + c * _i32(CH), CH)], ysems[b])

    xcopy(_i32(0), 0).start()
    for d in rdescs:
        d.wait()

    def outer(g, _):
        for b in range(2):
            c = g * _i32(2) + _i32(b)
            nxt = c + _i32(1)

            @pl.when(nxt < _i32(NCHUNK))
            def _start():
                xcopy(nxt, 1 - b).start()

            xcopy(c, b).wait()

            @pl.when(c >= _i32(2))
            def _drain():
                ycopy(c - _i32(2), b).wait()

            @plsc.parallel_loop(_i32(0), _i32(CH // L), _i32(1), unroll=8)
            def slab(k):
                v = xbuf[_i32(b), pl.ds(k * _i32(L), L)]
                ybuf[_i32(b), pl.ds(k * _i32(L), L)] = plsc.load_gather(
                    rank_v, [v])

            ycopy(c, b).start()
        return _

    _fori(NCHUNK // 2, outer, 0)
    for b in range(2):
        ycopy(_i32(NCHUNK - 2 + b), b).wait()


def kernel(x):
    x32 = x.astype(jnp.int32)
    part = _hist_kernel(x32)
    hist, tot = _merge_kernel(part)
    rank, pv, pc = _rank_kernel(hist, tot)
    vals_p, cnts_p = _final_kernel(tot, pv, pc)
    inv32 = _inverse_kernel(x32, rank)
    values = vals_p[:VOCAB].astype(jnp.uint32).astype(x.dtype)
    counts = cnts_p[:VOCAB].astype(jnp.uint32).astype(jnp.int64)
    inverse = inv32.astype(jnp.uint32).astype(jnp.int64)
    return (values, inverse, counts)


# speedup vs baseline: 1.0035x; 1.0035x over previous
import functools

import jax
import jax.numpy as jnp
from jax import lax
from jax.experimental import pallas as pl
from jax.experimental.pallas import tpu as pltpu
from jax.experimental.pallas import tpu_sc as plsc

N = 8388608
VOCAB = 100000
NC = 2
NS = 16
L = 16
NW = NC * NS
VP = 102400
S = VP // NW
SLABS = S // L
SP = 3264
VOUT = NW * SP
NWCH = N // NW
CH = 4096
NCHUNK = NWCH // CH

_mesh = plsc.VectorSubcoreMesh(core_axis_name="c", subcore_axis_name="s")
_params = pltpu.CompilerParams(needs_layout_passes=False)


def _i32(v):
    return jnp.int32(v)


def _fori(n, body, init):
    return lax.fori_loop(jnp.int32(0), jnp.int32(n) if isinstance(n, int) else n,
                         body, init)


def _wid():
    return lax.axis_index("c") * _i32(NS) + lax.axis_index("s")


_GDN = lax.GatherDimensionNumbers(
    offset_dims=(), collapsed_slice_dims=(0,), start_index_map=(0,))


def _dyngather(vec, idx):
    return lax.gather(vec, idx[:, None], _GDN, (1,),
                      mode=lax.GatherScatterMode.PROMISE_IN_BOUNDS)


def _splat(vec, lane):
    return _dyngather(vec, jnp.full((L,), lane, jnp.int32))


@functools.partial(
    pl.kernel,
    out_type=jax.ShapeDtypeStruct((NW, VP), jnp.int32),
    mesh=_mesh,
    compiler_params=_params,
    scratch_types=[
        pltpu.VMEM((VP,), jnp.int32),
        pltpu.VMEM((2, CH), jnp.int32),
        pltpu.SemaphoreType.DMA,
        pltpu.SemaphoreType.DMA,
    ],
)
def _hist_kernel(x_hbm, part_hbm, hist_v, xbuf, sem0, sem1):
    w = _wid()
    sems = (sem0, sem1)
    zero = jnp.zeros((L,), jnp.int32)

    def zbody(i, _):
        hist_v[pl.ds(i * _i32(L), L)] = zero
        return _

    _fori(VP // L, zbody, 0)

    base = w * _i32(NWCH)

    base = w * _i32(NWCH)

    def xcopy(c, b):
        return pltpu.make_async_copy(
            x_hbm.at[pl.ds(base + c * _i32(CH), CH)], xbuf.at[_i32(b)], sems[b])

    xcopy(_i32(0), 0).start()

    def outer(g, _):
        for b in range(2):
            c = g * _i32(2) + _i32(b)
            nxt = c + _i32(1)

            @pl.when(nxt < _i32(NCHUNK))
            def _start():
                xcopy(nxt, 1 - b).start()

            xcopy(c, b).wait()

            def slab(k, _2):
                k8 = k * _i32(8 * L)
                vs = [xbuf[_i32(b), pl.ds(k8 + _i32(u * L), L)] for u in range(8)]
                cms = [plsc.scan_count(v) for v in vs]
                for v, (cnt, last) in zip(vs, cms):
                    plsc.addupdate_scatter(hist_v, [v], cnt, mask=last)
                return _2

            _fori(CH // L // 8, slab, 0)
        return _

    _fori(NCHUNK // 2, outer, 0)
    pltpu.sync_copy(hist_v, part_hbm.at[w])


@functools.partial(
    pl.kernel,
    out_type=(
        jax.ShapeDtypeStruct((VP,), jnp.int32),
        jax.ShapeDtypeStruct((NW, L), jnp.int32),
    ),
    mesh=_mesh,
    compiler_params=_params,
    scratch_types=[
        pltpu.VMEM((NW, S), jnp.int32),
        pltpu.VMEM((S,), jnp.int32),
        pltpu.VMEM((L,), jnp.int32),
        pltpu.SemaphoreType.DMA,
    ],
)
def _merge_kernel(part_hbm, hist_hbm, tot_hbm, rows_v, acc_v, stage_v, sem):
    w = _wid()
    descs = []
    for j in range(NW):
        d = pltpu.make_async_copy(
            part_hbm.at[_i32(j), pl.ds(w * _i32(S), S)], rows_v.at[_i32(j)], sem)
        d.start()
        descs.append(d)
    for d in descs:
        d.wait()

    def slab(k, occ_tot):
        a = rows_v[_i32(0), pl.ds(k * _i32(L), L)]
        for j in range(1, NW):
            a = a + rows_v[_i32(j), pl.ds(k * _i32(L), L)]
        acc_v[pl.ds(k * _i32(L), L)] = a
        return occ_tot + (a > 0).astype(jnp.int32)

    occ_tot = _fori(SLABS, slab, jnp.zeros((L,), jnp.int32))
    total = jnp.sum(occ_tot, dtype=jnp.int32)
    stage_v[...] = jnp.zeros((L,), jnp.int32) + total
    pltpu.sync_copy(acc_v, hist_hbm.at[pl.ds(w * _i32(S), S)])
    pltpu.sync_copy(stage_v, tot_hbm.at[w])


@functools.partial(
    pl.kernel,
    out_type=(
        jax.ShapeDtypeStruct((VP,), jnp.int32),
        jax.ShapeDtypeStruct((NW, S), jnp.int32),
        jax.ShapeDtypeStruct((NW, S), jnp.int32),
    ),
    mesh=_mesh,
    compiler_params=_params,
    scratch_types=[
        pltpu.VMEM((NW, L), jnp.int32),
        pltpu.VMEM((S,), jnp.int32),
        pltpu.VMEM((S,), jnp.int32),
        pltpu.VMEM((S + L,), jnp.int32),
        pltpu.VMEM((S + L,), jnp.int32),
    ],
)
def _rank_kernel(hist_hbm, tot_hbm, rank_hbm, pv_hbm, pc_hbm,
                 tot_v, hs_v, rk_v, cv_v, cc_v):
    w = _wid()
    pltpu.sync_copy(tot_hbm, tot_v)
    pltpu.sync_copy(hist_hbm.at[pl.ds(w * _i32(S), S)], hs_v)

    zvec = jnp.zeros((L,), jnp.int32)
    off = zvec
    for j in range(NW):
        row = tot_v[_i32(j)]
        off = off + jnp.where(jnp.full((L,), j, jnp.int32) < w, row, zvec)

    iota = lax.iota(jnp.int32, L)
    vbase = w * _i32(S) + iota

    def slab(k, carry):
        h = hs_v[pl.ds(k * _i32(L), L)]
        occ = h > 0
        occ_i = occ.astype(jnp.int32)
        lidx = plsc.cumsum(occ_i)
        rank = off + (carry - _i32(1)) + lidx
        rk_v[pl.ds(k * _i32(L), L)] = rank
        vglob = vbase + k * _i32(L)
        plsc.store_compressed(cv_v.at[pl.ds(carry, L)], vglob, mask=occ)
        plsc.store_compressed(cc_v.at[pl.ds(carry, L)], h, mask=occ)
        return carry + jnp.sum(occ_i, dtype=jnp.int32)

    _fori(SLABS, slab, _i32(0))

    pltpu.sync_copy(rk_v, rank_hbm.at[pl.ds(w * _i32(S), S)])
    pltpu.sync_copy(cv_v.at[pl.ds(_i32(0), S)], pv_hbm.at[w])
    pltpu.sync_copy(cc_v.at[pl.ds(_i32(0), S)], pc_hbm.at[w])


@functools.partial(
    pl.kernel,
    out_type=(
        jax.ShapeDtypeStruct((VOUT,), jnp.int32),
        jax.ShapeDtypeStruct((VOUT,), jnp.int32),
    ),
    mesh=_mesh,
    compiler_params=_params,
    scratch_types=[
        pltpu.VMEM((NW, L), jnp.int32),
        pltpu.VMEM((NW * S,), jnp.int32),
        pltpu.VMEM((SP,), jnp.int32),
        pltpu.VMEM((SP,), jnp.int32),
        pltpu.SemaphoreType.DMA,
    ],
)
def _final_kernel(tot_hbm, pv_hbm, pc_hbm, vals_hbm, cnts_hbm,
                  tot_v, buf_v, src_v, out_v, sem):
    w = _wid()
    pltpu.sync_copy(tot_hbm, tot_v)

    iota = lax.iota(jnp.int32, L)
    zvec = jnp.zeros((L,), jnp.int32)
    off_lo = zvec
    off_hi = zvec
    for j in range(NW):
        row = tot_v[_i32(j)]
        if j < 15:
            off_lo = off_lo + jnp.where(iota > j, row, zvec)
        if j < 16:
            off_hi = off_hi + row
        else:
            off_hi = off_hi + jnp.where(iota > j - 16, row, zvec)
    t31 = tot_v[_i32(NW - 1)]
    nuq = _splat(off_hi, 15) + t31

    b0 = w * _i32(SP)

    bnd = [_splat(off_lo, k) for k in range(1, 16)] + \
          [_splat(off_hi, k) for k in range(16)] + [nuq]

    def mapslab(k, _):
        r = b0 + k * _i32(L) + iota
        p = zvec
        for bk in bnd:
            p = p + (bk <= r).astype(jnp.int32)
        pcl = jnp.minimum(p, _i32(NW - 1))
        ia = jnp.minimum(pcl, _i32(15))
        ib = jnp.clip(pcl - _i32(16), _i32(0), _i32(15))
        offp = jnp.where(pcl < _i32(16),
                         _dyngather(off_lo, ia),
                         _dyngather(off_hi, ib))
        src = pcl * _i32(S) + (r - offp)
        src = jnp.clip(src, _i32(0), _i32(NW * S - 1))
        src_v[pl.ds(k * _i32(L), L)] = src
        return _

    _fori(SP // L, mapslab, 0)

    for src_hbm, dst_hbm in ((pv_hbm, vals_hbm), (pc_hbm, cnts_hbm)):
        descs = []
        for i in range(NW):
            j = (w + _i32(i)) % _i32(NW)
            d = pltpu.make_async_copy(
                src_hbm.at[j], buf_v.at[pl.ds(j * _i32(S), S)], sem)
            d.start()
            descs.append(d)
        for d in descs:
            d.wait()

        def gslab(k, _):
            r = b0 + k * _i32(L) + iota
            idx = src_v[pl.ds(k * _i32(L), L)]
            g = plsc.load_gather(buf_v, [idx])
            out_v[pl.ds(k * _i32(L), L)] = jnp.where(r < nuq, g, zvec)
            return _

        _fori(SP // L, gslab, 0)
        pltpu.sync_copy(out_v, dst_hbm.at[pl.ds(b0, SP)])


@functools.partial(
    pl.kernel,
    out_type=jax.ShapeDtypeStruct((N,), jnp.int32),
    mesh=_mesh,
    compiler_params=_params,
    scratch_types=[
        pltpu.VMEM((VP,), jnp.int32),
        pltpu.VMEM((2, CH), jnp.int32),
        pltpu.VMEM((2, CH), jnp.int32),
        pltpu.SemaphoreType.DMA,
        pltpu.SemaphoreType.DMA,
        pltpu.SemaphoreType.DMA,
        pltpu.SemaphoreType.DMA,
        pltpu.SemaphoreType.DMA,
    ],
)
def _inverse_kernel(x_hbm, rank_hbm, inv_hbm, rank_v, xbuf, ybuf,
                    xs0, xs1, ys0, ys1, rsem):
    w = _wid()
    rdescs = []
    for i in range(NW):
        j = (w + _i32(i)) % _i32(NW)
        d = pltpu.make_async_copy(
            rank_hbm.at[pl.ds(j * _i32(S), S)], rank_v.at[pl.ds(j * _i32(S), S)],
            rsem)
        d.start()
        rdescs.append(d)

    base = w * _i32(NWCH)
    xsems = (xs0, xs1)
    ysems = (ys0, ys1)

    def xcopy(c, b):
        return pltpu.make_async_copy(
            x_hbm.at[pl.ds(base + c * _i32(CH), CH)], xbuf.at[_i32(b)], xsems[b])

    def ycopy(c, b):
        return pltpu.make_async_copy(
            ybuf.at[_i32(b)], inv_hbm.at[pl.ds(base + c * _i32(CH), CH)], ysems[b])

    xcopy(_i32(0), 0).start()
    for d in rdescs:
        d.wait()

    def outer(g, _):
        for b in range(2):
            c = g * _i32(2) + _i32(b)
            nxt = c + _i32(1)

            @pl.when(nxt < _i32(NCHUNK))
            def _start():
                xcopy(nxt, 1 - b).start()

            xcopy(c, b).wait()

            @pl.when(c >= _i32(2))
            def _drain():
                ycopy(c - _i32(2), b).wait()

            @plsc.parallel_loop(_i32(0), _i32(CH // L), _i32(1), unroll=8)
            def slab(k):
                v = xbuf[_i32(b), pl.ds(k * _i32(L), L)]
                ybuf[_i32(b), pl.ds(k * _i32(L), L)] = plsc.load_gather(
                    rank_v, [v])

            ycopy(c, b).start()
        return _

    _fori(NCHUNK // 2, outer, 0)
    for b in range(2):
        ycopy(_i32(NCHUNK - 2 + b), b).wait()


def kernel(x):
    x32 = x.astype(jnp.int32)
    part = _hist_kernel(x32)
    hist, tot = _merge_kernel(part)
    rank, pv, pc = _rank_kernel(hist, tot)
    vals_p, cnts_p = _final_kernel(tot, pv, pc)
    inv32 = _inverse_kernel(x32, rank)
    values = vals_p[:VOCAB].astype(x.dtype)
    counts = cnts_p[:VOCAB].astype(jnp.int64)
    inverse = inv32.astype(jnp.int64)
    return (values, inverse, counts)
